# baseline (speedup 1.0000x reference)
import functools

import jax
import jax.numpy as jnp
from jax import lax
from jax.experimental import pallas as pl
from jax.experimental.pallas import tpu as pltpu
from jax.experimental.pallas import tpu_sc as plsc

_B = 16384
_D = 32
_PACK = 4
_LINE = _D * _PACK
_NC, _NS = 2, 16
_NW = _NC * _NS
_BPW = _B // _NW
_CHUNK = 64
_NCHUNK = _BPW // _CHUNK
_NBUF = 2

_BLK = 2048

_V = 1000000
_VLINE = _V // _PACK
_RPW = 2048
_RPB = _RPW // _PACK


def _repack_body(t_ref, o_ref):
    x = t_ref[...]
    y = jnp.transpose(x.reshape(_D, _RPB, _PACK), (1, 2, 0))
    o_ref[...] = y.reshape(_RPB, _LINE)


def _tc_repack(tT):
    nblk = (_V + _RPW - 1) // _RPW
    return pl.pallas_call(
        _repack_body,
        grid=(nblk,),
        in_specs=[pl.BlockSpec((_D, _RPW), lambda i: (0, i))],
        out_specs=pl.BlockSpec((_RPB, _LINE), lambda i: (i, 0)),
        out_shape=jax.ShapeDtypeStruct((_VLINE, _LINE), jnp.float32),
    )(tT)


def _sc_gather4(u_idx3, i_idx3, t_ug, t_ig, t_um, t_im):
    mesh = plsc.VectorSubcoreMesh(core_axis_name="c", subcore_axis_name="s")
    out4 = jax.ShapeDtypeStruct((_NW, _NCHUNK, _CHUNK, _LINE), jnp.float32)

    @functools.partial(
        pl.kernel,
        mesh=mesh,
        out_type=[out4, out4, out4, out4],
        compiler_params=pltpu.CompilerParams(use_tc_tiling_on_sc=True),
        scratch_types=[
            pltpu.VMEM((_NCHUNK, _CHUNK), jnp.int32),
            pltpu.VMEM((_NCHUNK, _CHUNK), jnp.int32),
            pltpu.VMEM((_NBUF, _CHUNK, _LINE), jnp.float32),
            pltpu.VMEM((_NBUF, _CHUNK, _LINE), jnp.float32),
            pltpu.VMEM((_NBUF, _CHUNK, _LINE), jnp.float32),
            pltpu.VMEM((_NBUF, _CHUNK, _LINE), jnp.float32),
            pltpu.SemaphoreType.DMA,
        ],
    )
    def k(uidx_hbm, iidx_hbm, ug_hbm, ig_hbm, um_hbm, im_hbm,
          o_ug, o_ig, o_um, o_im,
          uix_v, iix_v, r_ug, r_ig, r_um, r_im, sem):
        wid = lax.axis_index("s") * _NC + lax.axis_index("c")
        pltpu.sync_copy(uidx_hbm.at[wid], uix_v)
        pltpu.sync_copy(iidx_hbm.at[wid], iix_v)
        bufs = (r_ug, r_ig, r_um, r_im)
        outs = (o_ug, o_ig, o_um, o_im)
        tabs = (ug_hbm, ig_hbm, um_hbm, im_hbm)
        ixs = (uix_v, iix_v, uix_v, iix_v)

        def fire(c):
            b = c % _NBUF
            return [pltpu.async_copy(tabs[t].at[ixs[t].at[c]], bufs[t].at[b], sem)
                    for t in range(4)]

        pending = {c: fire(c) for c in range(_NBUF)}
        for c in range(_NCHUNK):
            for cp in pending.pop(c):
                cp.wait()
            b = c % _NBUF
            for t in range(4):
                pltpu.sync_copy(bufs[t].at[b], outs[t].at[wid, c])
            if c + _NBUF < _NCHUNK:
                pending[c + _NBUF] = fire(c + _NBUF)

    return k(u_idx3, i_idx3, t_ug, t_ig, t_um, t_im)


def _select32(wide, masks):
    acc = masks[0] * wide[:, 0:_D]
    for s in range(1, _PACK):
        acc += masks[s] * wide[:, s * _D:(s + 1) * _D]
    return acc


def _mlp_body(ug_ref, ig_ref, um_ref, im_ref, usel_ref, isel_ref,
              w1a_ref, w1b_ref, b1_ref, w2_ref, b2_ref, wpa_ref, wpb_ref,
              bp_ref, o_ref):
    um_sel = usel_ref[...]
    im_sel = isel_ref[...]
    umask = [um_sel[:, s:s + 1] for s in range(_PACK)]
    imask = [im_sel[:, s:s + 1] for s in range(_PACK)]
    ug = _select32(ug_ref[...], umask)
    ig = _select32(ig_ref[...], imask)
    um = _select32(um_ref[...], umask)
    im = _select32(im_ref[...], imask)
    h1 = jnp.dot(um, w1a_ref[...], preferred_element_type=jnp.float32)
    h1 += jnp.dot(im, w1b_ref[...], preferred_element_type=jnp.float32)
    h1 = jnp.maximum(h1 + b1_ref[...], 0.0)
    h2 = jnp.dot(h1, w2_ref[...], preferred_element_type=jnp.float32)
    h2 = jnp.maximum(h2 + b2_ref[...], 0.0)
    g = ug * ig
    p = (jnp.sum(g * wpa_ref[...], axis=1, keepdims=True)
         + jnp.sum(h2 * wpb_ref[...], axis=1, keepdims=True)
         + bp_ref[...])
    o_ref[...] = jax.nn.sigmoid(p)


def _tc_mlp(ug, ig, um, im, usel, isel, w1a, w1b, b1r, w2, b2r, wpa, wpb, bpr):
    wide_spec = pl.BlockSpec((_BLK, _LINE), lambda i: (i, 0))
    sel_spec = pl.BlockSpec((_BLK, _PACK), lambda i: (i, 0))

    def full(shape):
        return pl.BlockSpec(shape, lambda i: (0, 0))

    return pl.pallas_call(
        _mlp_body,
        grid=(_B // _BLK,),
        in_specs=[
            wide_spec, wide_spec, wide_spec, wide_spec,
            sel_spec, sel_spec,
            full((_D, 32)), full((_D, 32)), full((1, 32)),
            full((32, 16)), full((1, 16)),
            full((1, _D)), full((1, 16)), full((1, 1)),
        ],
        out_specs=pl.BlockSpec((_BLK, 1), lambda i: (i, 0)),
        out_shape=jax.ShapeDtypeStruct((_B, 1), jnp.float32),
    )(ug, ig, um, im, usel, isel, w1a, w1b, b1r, w2, b2r, wpa, wpb, bpr)


def kernel(user_indices, item_indices, embed_user_GMF, embed_item_GMF,
           embed_user_MLP, embed_item_MLP, W1, b1, W2, b2, Wp, bp):
    ui = user_indices.astype(jnp.int32)
    ii = item_indices.astype(jnp.int32)
    u3 = (ui // _PACK).reshape(_NW, _NCHUNK, _CHUNK)
    i3 = (ii // _PACK).reshape(_NW, _NCHUNK, _CHUNK)
    tables = [_tc_repack(t.T) for t in
              (embed_user_GMF, embed_item_GMF, embed_user_MLP, embed_item_MLP)]
    ug, ig, um, im = _sc_gather4(u3, i3, *tables)
    ug = ug.reshape(_B, _LINE)
    ig = ig.reshape(_B, _LINE)
    um = um.reshape(_B, _LINE)
    im = im.reshape(_B, _LINE)
    usel = jax.nn.one_hot(ui % _PACK, _PACK, dtype=jnp.float32)
    isel = jax.nn.one_hot(ii % _PACK, _PACK, dtype=jnp.float32)
    w1a, w1b = W1[:_D], W1[_D:]
    wpa = Wp[:_D, 0].reshape(1, _D)
    wpb = Wp[_D:, 0].reshape(1, 16)
    out = _tc_mlp(ug, ig, um, im, usel, isel, w1a, w1b, b1.reshape(1, 32),
                  W2, b2.reshape(1, 16), wpa, wpb, bp.reshape(1, 1))
    return out.reshape(-1)


# speedup vs baseline: 19.3178x; 19.3178x over previous
import functools

import jax
import jax.numpy as jnp
from jax import lax
from jax.experimental import pallas as pl
from jax.experimental.pallas import tpu as pltpu
from jax.experimental.pallas import tpu_sc as plsc

_B = 16384
_D = 32
_PACK = 4
_LINE = _D * _PACK
_NC, _NS = 2, 16
_NW = _NC * _NS
_BPW = _B // _NW
_CHUNK = 64
_NCHUNK = _BPW // _CHUNK
_NBUF = 2

_BLK = 2048

_V = 1000000
_RPW = 4096
_NJ = 62
_S = _RPW * _NJ


def _repack_body(t0_ref, t1_ref, t2_ref, t3_ref, o_ref):
    x = jnp.concatenate(
        [t0_ref[...], t1_ref[...], t2_ref[...], t3_ref[...]], axis=0)
    o_ref[...] = x.T


def _tc_repack(tT):
    last_blk = (_V + _RPW - 1) // _RPW - 1

    def in_spec(s):
        return pl.BlockSpec(
            (_D, _RPW),
            lambda j, s=s: (0, jnp.minimum(_NJ * s + j, last_blk)))

    return pl.pallas_call(
        _repack_body,
        grid=(_NJ,),
        in_specs=[in_spec(0), in_spec(1), in_spec(2), in_spec(3)],
        out_specs=pl.BlockSpec((_RPW, _LINE), lambda j: (j, 0)),
        out_shape=jax.ShapeDtypeStruct((_S, _LINE), jnp.float32),
    )(tT, tT, tT, tT)


def _sc_gather4(u_idx3, i_idx3, t_ug, t_ig, t_um, t_im):
    mesh = plsc.VectorSubcoreMesh(core_axis_name="c", subcore_axis_name="s")
    out4 = jax.ShapeDtypeStruct((_NW, _NCHUNK, _CHUNK, _LINE), jnp.float32)

    @functools.partial(
        pl.kernel,
        mesh=mesh,
        out_type=[out4, out4, out4, out4],
        compiler_params=pltpu.CompilerParams(use_tc_tiling_on_sc=True),
        scratch_types=[
            pltpu.VMEM((_NCHUNK, _CHUNK), jnp.int32),
            pltpu.VMEM((_NCHUNK, _CHUNK), jnp.int32),
            pltpu.VMEM((_NBUF, _CHUNK, _LINE), jnp.float32),
            pltpu.VMEM((_NBUF, _CHUNK, _LINE), jnp.float32),
            pltpu.VMEM((_NBUF, _CHUNK, _LINE), jnp.float32),
            pltpu.VMEM((_NBUF, _CHUNK, _LINE), jnp.float32),
            pltpu.SemaphoreType.DMA,
        ],
    )
    def k(uidx_hbm, iidx_hbm, ug_hbm, ig_hbm, um_hbm, im_hbm,
          o_ug, o_ig, o_um, o_im,
          uix_v, iix_v, r_ug, r_ig, r_um, r_im, sem):
        wid = lax.axis_index("s") * _NC + lax.axis_index("c")
        pltpu.sync_copy(uidx_hbm.at[wid], uix_v)
        pltpu.sync_copy(iidx_hbm.at[wid], iix_v)
        bufs = (r_ug, r_ig, r_um, r_im)
        outs = (o_ug, o_ig, o_um, o_im)
        tabs = (ug_hbm, ig_hbm, um_hbm, im_hbm)
        ixs = (uix_v, iix_v, uix_v, iix_v)

        def fire(c):
            b = c % _NBUF
            return [pltpu.async_copy(tabs[t].at[ixs[t].at[c]], bufs[t].at[b], sem)
                    for t in range(4)]

        pending = {c: fire(c) for c in range(_NBUF)}
        for c in range(_NCHUNK):
            for cp in pending.pop(c):
                cp.wait()
            b = c % _NBUF
            for t in range(4):
                pltpu.sync_copy(bufs[t].at[b], outs[t].at[wid, c])
            if c + _NBUF < _NCHUNK:
                pending[c + _NBUF] = fire(c + _NBUF)

    return k(u_idx3, i_idx3, t_ug, t_ig, t_um, t_im)


def _select32(wide, masks):
    acc = masks[0] * wide[:, 0:_D]
    for s in range(1, _PACK):
        acc += masks[s] * wide[:, s * _D:(s + 1) * _D]
    return acc


def _mlp_body(ug_ref, ig_ref, um_ref, im_ref, usel_ref, isel_ref,
              w1a_ref, w1b_ref, b1_ref, w2_ref, b2_ref, wpa_ref, wpb_ref,
              bp_ref, o_ref):
    um_sel = usel_ref[...]
    im_sel = isel_ref[...]
    umask = [um_sel[:, s:s + 1] for s in range(_PACK)]
    imask = [im_sel[:, s:s + 1] for s in range(_PACK)]
    ug = _select32(ug_ref[...], umask)
    ig = _select32(ig_ref[...], imask)
    um = _select32(um_ref[...], umask)
    im = _select32(im_ref[...], imask)
    h1 = jnp.dot(um, w1a_ref[...], preferred_element_type=jnp.float32)
    h1 += jnp.dot(im, w1b_ref[...], preferred_element_type=jnp.float32)
    h1 = jnp.maximum(h1 + b1_ref[...], 0.0)
    h2 = jnp.dot(h1, w2_ref[...], preferred_element_type=jnp.float32)
    h2 = jnp.maximum(h2 + b2_ref[...], 0.0)
    g = ug * ig
    p = (jnp.sum(g * wpa_ref[...], axis=1, keepdims=True)
         + jnp.sum(h2 * wpb_ref[...], axis=1, keepdims=True)
         + bp_ref[...])
    o_ref[...] = jax.nn.sigmoid(p)


def _tc_mlp(ug, ig, um, im, usel, isel, w1a, w1b, b1r, w2, b2r, wpa, wpb, bpr):
    wide_spec = pl.BlockSpec((_BLK, _LINE), lambda i: (i, 0))
    sel_spec = pl.BlockSpec((_BLK, _PACK), lambda i: (i, 0))

    def full(shape):
        return pl.BlockSpec(shape, lambda i: (0, 0))

    return pl.pallas_call(
        _mlp_body,
        grid=(_B // _BLK,),
        in_specs=[
            wide_spec, wide_spec, wide_spec, wide_spec,
            sel_spec, sel_spec,
            full((_D, 32)), full((_D, 32)), full((1, 32)),
            full((32, 16)), full((1, 16)),
            full((1, _D)), full((1, 16)), full((1, 1)),
        ],
        out_specs=pl.BlockSpec((_BLK, 1), lambda i: (i, 0)),
        out_shape=jax.ShapeDtypeStruct((_B, 1), jnp.float32),
    )(ug, ig, um, im, usel, isel, w1a, w1b, b1r, w2, b2r, wpa, wpb, bpr)


def kernel(user_indices, item_indices, embed_user_GMF, embed_item_GMF,
           embed_user_MLP, embed_item_MLP, W1, b1, W2, b2, Wp, bp):
    ui = user_indices.astype(jnp.int32)
    ii = item_indices.astype(jnp.int32)
    u3 = (ui % _S).reshape(_NW, _NCHUNK, _CHUNK)
    i3 = (ii % _S).reshape(_NW, _NCHUNK, _CHUNK)
    tables = [_tc_repack(t.T) for t in
              (embed_user_GMF, embed_item_GMF, embed_user_MLP, embed_item_MLP)]
    ug, ig, um, im = _sc_gather4(u3, i3, *tables)
    ug = ug.reshape(_B, _LINE)
    ig = ig.reshape(_B, _LINE)
    um = um.reshape(_B, _LINE)
    im = im.reshape(_B, _LINE)
    usel = jax.nn.one_hot(ui // _S, _PACK, dtype=jnp.float32)
    isel = jax.nn.one_hot(ii // _S, _PACK, dtype=jnp.float32)
    w1a, w1b = W1[:_D], W1[_D:]
    wpa = Wp[:_D, 0].reshape(1, _D)
    wpb = Wp[_D:, 0].reshape(1, 16)
    out = _tc_mlp(ug, ig, um, im, usel, isel, w1a, w1b, b1.reshape(1, 32),
                  W2, b2.reshape(1, 16), wpa, wpb, bp.reshape(1, 1))
    return out.reshape(-1)


# speedup vs baseline: 19.3181x; 1.0000x over previous
import functools

import jax
import jax.numpy as jnp
from jax import lax
from jax.experimental import pallas as pl
from jax.experimental.pallas import tpu as pltpu
from jax.experimental.pallas import tpu_sc as plsc

_B = 16384
_D = 32
_PACK = 4
_LINE = _D * _PACK
_NC, _NS = 2, 16
_NW = _NC * _NS
_BPW = _B // _NW
_CHUNK = 64
_NCHUNK = _BPW // _CHUNK
_NBUF = 2

_BLK = 2048

_V = 1000000
_RPW = 4096
_NJ = 62
_S = _RPW * _NJ


def _repack_body(t0_ref, t1_ref, t2_ref, t3_ref, o_ref):
    x = jnp.concatenate(
        [t0_ref[...], t1_ref[...], t2_ref[...], t3_ref[...]], axis=0)
    o_ref[...] = x.T


def _tc_repack(tT):
    last_blk = (_V + _RPW - 1) // _RPW - 1

    def in_spec(s):
        return pl.BlockSpec(
            (_D, _RPW),
            lambda j, s=s: (0, jnp.minimum(_NJ * s + j, last_blk)))

    return pl.pallas_call(
        _repack_body,
        grid=(_NJ,),
        in_specs=[in_spec(0), in_spec(1), in_spec(2), in_spec(3)],
        out_specs=pl.BlockSpec((_RPW, _LINE), lambda j: (j, 0)),
        out_shape=jax.ShapeDtypeStruct((_S, _LINE), jnp.float32),
        compiler_params=pltpu.CompilerParams(
            dimension_semantics=("parallel",)),
    )(tT, tT, tT, tT)


def _sc_gather4(u_idx3, i_idx3, t_ug, t_ig, t_um, t_im):
    mesh = plsc.VectorSubcoreMesh(core_axis_name="c", subcore_axis_name="s")
    out4 = jax.ShapeDtypeStruct((_NW, _NCHUNK, _CHUNK, _LINE), jnp.float32)

    @functools.partial(
        pl.kernel,
        mesh=mesh,
        out_type=[out4, out4, out4, out4],
        compiler_params=pltpu.CompilerParams(use_tc_tiling_on_sc=True),
        scratch_types=[
            pltpu.VMEM((_NCHUNK, _CHUNK), jnp.int32),
            pltpu.VMEM((_NCHUNK, _CHUNK), jnp.int32),
            pltpu.VMEM((_NBUF, _CHUNK, _LINE), jnp.float32),
            pltpu.VMEM((_NBUF, _CHUNK, _LINE), jnp.float32),
            pltpu.VMEM((_NBUF, _CHUNK, _LINE), jnp.float32),
            pltpu.VMEM((_NBUF, _CHUNK, _LINE), jnp.float32),
            pltpu.SemaphoreType.DMA,
        ],
    )
    def k(uidx_hbm, iidx_hbm, ug_hbm, ig_hbm, um_hbm, im_hbm,
          o_ug, o_ig, o_um, o_im,
          uix_v, iix_v, r_ug, r_ig, r_um, r_im, sem):
        wid = lax.axis_index("s") * _NC + lax.axis_index("c")
        pltpu.sync_copy(uidx_hbm.at[wid], uix_v)
        pltpu.sync_copy(iidx_hbm.at[wid], iix_v)
        bufs = (r_ug, r_ig, r_um, r_im)
        outs = (o_ug, o_ig, o_um, o_im)
        tabs = (ug_hbm, ig_hbm, um_hbm, im_hbm)
        ixs = (uix_v, iix_v, uix_v, iix_v)

        def fire(c):
            b = c % _NBUF
            return [pltpu.async_copy(tabs[t].at[ixs[t].at[c]], bufs[t].at[b], sem)
                    for t in range(4)]

        pending = {c: fire(c) for c in range(_NBUF)}
        for c in range(_NCHUNK):
            for cp in pending.pop(c):
                cp.wait()
            b = c % _NBUF
            for t in range(4):
                pltpu.sync_copy(bufs[t].at[b], outs[t].at[wid, c])
            if c + _NBUF < _NCHUNK:
                pending[c + _NBUF] = fire(c + _NBUF)

    return k(u_idx3, i_idx3, t_ug, t_ig, t_um, t_im)


def _select32(wide, masks):
    acc = masks[0] * wide[:, 0:_D]
    for s in range(1, _PACK):
        acc += masks[s] * wide[:, s * _D:(s + 1) * _D]
    return acc


def _mlp_body(ug_ref, ig_ref, um_ref, im_ref, usel_ref, isel_ref,
              w1a_ref, w1b_ref, b1_ref, w2_ref, b2_ref, wpa_ref, wpb_ref,
              bp_ref, o_ref):
    um_sel = usel_ref[...]
    im_sel = isel_ref[...]
    umask = [um_sel[:, s:s + 1] for s in range(_PACK)]
    imask = [im_sel[:, s:s + 1] for s in range(_PACK)]
    ug = _select32(ug_ref[...], umask)
    ig = _select32(ig_ref[...], imask)
    um = _select32(um_ref[...], umask)
    im = _select32(im_ref[...], imask)
    h1 = jnp.dot(um, w1a_ref[...], preferred_element_type=jnp.float32)
    h1 += jnp.dot(im, w1b_ref[...], preferred_element_type=jnp.float32)
    h1 = jnp.maximum(h1 + b1_ref[...], 0.0)
    h2 = jnp.dot(h1, w2_ref[...], preferred_element_type=jnp.float32)
    h2 = jnp.maximum(h2 + b2_ref[...], 0.0)
    g = ug * ig
    p = (jnp.sum(g * wpa_ref[...], axis=1, keepdims=True)
         + jnp.sum(h2 * wpb_ref[...], axis=1, keepdims=True)
         + bp_ref[...])
    o_ref[...] = jax.nn.sigmoid(p)


def _tc_mlp(ug, ig, um, im, usel, isel, w1a, w1b, b1r, w2, b2r, wpa, wpb, bpr):
    wide_spec = pl.BlockSpec((_BLK, _LINE), lambda i: (i, 0))
    sel_spec = pl.BlockSpec((_BLK, _PACK), lambda i: (i, 0))

    def full(shape):
        return pl.BlockSpec(shape, lambda i: (0, 0))

    return pl.pallas_call(
        _mlp_body,
        grid=(_B // _BLK,),
        in_specs=[
            wide_spec, wide_spec, wide_spec, wide_spec,
            sel_spec, sel_spec,
            full((_D, 32)), full((_D, 32)), full((1, 32)),
            full((32, 16)), full((1, 16)),
            full((1, _D)), full((1, 16)), full((1, 1)),
        ],
        out_specs=pl.BlockSpec((_BLK, 1), lambda i: (i, 0)),
        out_shape=jax.ShapeDtypeStruct((_B, 1), jnp.float32),
        compiler_params=pltpu.CompilerParams(
            dimension_semantics=("parallel",)),
    )(ug, ig, um, im, usel, isel, w1a, w1b, b1r, w2, b2r, wpa, wpb, bpr)


def kernel(user_indices, item_indices, embed_user_GMF, embed_item_GMF,
           embed_user_MLP, embed_item_MLP, W1, b1, W2, b2, Wp, bp):
    ui = user_indices.astype(jnp.int32)
    ii = item_indices.astype(jnp.int32)
    u3 = (ui % _S).reshape(_NW, _NCHUNK, _CHUNK)
    i3 = (ii % _S).reshape(_NW, _NCHUNK, _CHUNK)
    tables = [_tc_repack(t.T) for t in
              (embed_user_GMF, embed_item_GMF, embed_user_MLP, embed_item_MLP)]
    ug, ig, um, im = _sc_gather4(u3, i3, *tables)
    ug = ug.reshape(_B, _LINE)
    ig = ig.reshape(_B, _LINE)
    um = um.reshape(_B, _LINE)
    im = im.reshape(_B, _LINE)
    usel = jax.nn.one_hot(ui // _S, _PACK, dtype=jnp.float32)
    isel = jax.nn.one_hot(ii // _S, _PACK, dtype=jnp.float32)
    w1a, w1b = W1[:_D], W1[_D:]
    wpa = Wp[:_D, 0].reshape(1, _D)
    wpb = Wp[_D:, 0].reshape(1, 16)
    out = _tc_mlp(ug, ig, um, im, usel, isel, w1a, w1b, b1.reshape(1, 32),
                  W2, b2.reshape(1, 16), wpa, wpb, bp.reshape(1, 1))
    return out.reshape(-1)


# speedup vs baseline: 22.6222x; 1.1710x over previous
import functools

import jax
import jax.numpy as jnp
from jax import lax
from jax.experimental import pallas as pl
from jax.experimental.pallas import tpu as pltpu
from jax.experimental.pallas import tpu_sc as plsc

_B = 16384
_D = 32
_PACK = 4
_LINE = _D * _PACK
_NC, _NS = 2, 16
_NW = _NC * _NS
_BPW = _B // _NW
_CHUNK = 64
_NCHUNK = _BPW // _CHUNK
_NBUF = 2

_BLK = 2048

_V = 1000000
_RPW = 4096
_NJ = 62
_S = _RPW * _NJ


def _repack_body(*refs):
    in_refs, out_refs = refs[:16], refs[16:]
    for t in range(4):
        x = jnp.concatenate([in_refs[4 * t + s][...] for s in range(4)],
                            axis=0)
        out_refs[t][...] = x.T


def _tc_repack4(tT0, tT1, tT2, tT3):
    last_blk = (_V + _RPW - 1) // _RPW - 1

    def in_spec(s):
        return pl.BlockSpec(
            (_D, _RPW),
            lambda j, s=s: (0, jnp.minimum(_NJ * s + j, last_blk)))

    out4 = jax.ShapeDtypeStruct((_S, _LINE), jnp.float32)
    return pl.pallas_call(
        _repack_body,
        grid=(_NJ,),
        in_specs=[in_spec(s) for _ in range(4) for s in range(4)],
        out_specs=[pl.BlockSpec((_RPW, _LINE), lambda j: (j, 0))] * 4,
        out_shape=[out4, out4, out4, out4],
        compiler_params=pltpu.CompilerParams(
            dimension_semantics=("parallel",)),
    )(*[t for t in (tT0, tT1, tT2, tT3) for _ in range(4)])


def _sc_gather4(u_idx3, i_idx3, t_ug, t_ig, t_um, t_im):
    mesh = plsc.VectorSubcoreMesh(core_axis_name="c", subcore_axis_name="s")
    out4 = jax.ShapeDtypeStruct((_NW, _NCHUNK, _CHUNK, _LINE), jnp.float32)

    @functools.partial(
        pl.kernel,
        mesh=mesh,
        out_type=[out4, out4, out4, out4],
        compiler_params=pltpu.CompilerParams(use_tc_tiling_on_sc=True),
        scratch_types=[
            pltpu.VMEM((_NCHUNK, _CHUNK), jnp.int32),
            pltpu.VMEM((_NCHUNK, _CHUNK), jnp.int32),
            pltpu.VMEM((_NBUF, _CHUNK, _LINE), jnp.float32),
            pltpu.VMEM((_NBUF, _CHUNK, _LINE), jnp.float32),
            pltpu.VMEM((_NBUF, _CHUNK, _LINE), jnp.float32),
            pltpu.VMEM((_NBUF, _CHUNK, _LINE), jnp.float32),
            pltpu.SemaphoreType.DMA,
        ],
    )
    def k(uidx_hbm, iidx_hbm, ug_hbm, ig_hbm, um_hbm, im_hbm,
          o_ug, o_ig, o_um, o_im,
          uix_v, iix_v, r_ug, r_ig, r_um, r_im, sem):
        wid = lax.axis_index("s") * _NC + lax.axis_index("c")
        pltpu.sync_copy(uidx_hbm.at[wid], uix_v)
        pltpu.sync_copy(iidx_hbm.at[wid], iix_v)
        bufs = (r_ug, r_ig, r_um, r_im)
        outs = (o_ug, o_ig, o_um, o_im)
        tabs = (ug_hbm, ig_hbm, um_hbm, im_hbm)
        ixs = (uix_v, iix_v, uix_v, iix_v)

        def fire(c):
            b = c % _NBUF
            return [pltpu.async_copy(tabs[t].at[ixs[t].at[c]], bufs[t].at[b], sem)
                    for t in range(4)]

        pending = {c: fire(c) for c in range(_NBUF)}
        for c in range(_NCHUNK):
            for cp in pending.pop(c):
                cp.wait()
            b = c % _NBUF
            for t in range(4):
                pltpu.sync_copy(bufs[t].at[b], outs[t].at[wid, c])
            if c + _NBUF < _NCHUNK:
                pending[c + _NBUF] = fire(c + _NBUF)

    return k(u_idx3, i_idx3, t_ug, t_ig, t_um, t_im)


def _select32(wide, masks):
    acc = masks[0] * wide[:, 0:_D]
    for s in range(1, _PACK):
        acc += masks[s] * wide[:, s * _D:(s + 1) * _D]
    return acc


def _mlp_body(ug_ref, ig_ref, um_ref, im_ref, usel_ref, isel_ref,
              w1a_ref, w1b_ref, b1_ref, w2_ref, b2_ref, wpa_ref, wpb_ref,
              bp_ref, o_ref):
    um_sel = usel_ref[...]
    im_sel = isel_ref[...]
    umask = [um_sel[:, s:s + 1] for s in range(_PACK)]
    imask = [im_sel[:, s:s + 1] for s in range(_PACK)]
    ug = _select32(ug_ref[...], umask)
    ig = _select32(ig_ref[...], imask)
    um = _select32(um_ref[...], umask)
    im = _select32(im_ref[...], imask)
    h1 = jnp.dot(um, w1a_ref[...], preferred_element_type=jnp.float32)
    h1 += jnp.dot(im, w1b_ref[...], preferred_element_type=jnp.float32)
    h1 = jnp.maximum(h1 + b1_ref[...], 0.0)
    h2 = jnp.dot(h1, w2_ref[...], preferred_element_type=jnp.float32)
    h2 = jnp.maximum(h2 + b2_ref[...], 0.0)
    g = ug * ig
    p = (jnp.sum(g * wpa_ref[...], axis=1, keepdims=True)
         + jnp.sum(h2 * wpb_ref[...], axis=1, keepdims=True)
         + bp_ref[...])
    o_ref[...] = jax.nn.sigmoid(p)


def _tc_mlp(ug, ig, um, im, usel, isel, w1a, w1b, b1r, w2, b2r, wpa, wpb, bpr):
    wide_spec = pl.BlockSpec((_BLK, _LINE), lambda i: (i, 0))
    sel_spec = pl.BlockSpec((_BLK, _PACK), lambda i: (i, 0))

    def full(shape):
        return pl.BlockSpec(shape, lambda i: (0, 0))

    return pl.pallas_call(
        _mlp_body,
        grid=(_B // _BLK,),
        in_specs=[
            wide_spec, wide_spec, wide_spec, wide_spec,
            sel_spec, sel_spec,
            full((_D, 32)), full((_D, 32)), full((1, 32)),
            full((32, 16)), full((1, 16)),
            full((1, _D)), full((1, 16)), full((1, 1)),
        ],
        out_specs=pl.BlockSpec((_BLK, 1), lambda i: (i, 0)),
        out_shape=jax.ShapeDtypeStruct((_B, 1), jnp.float32),
        compiler_params=pltpu.CompilerParams(
            dimension_semantics=("parallel",)),
    )(ug, ig, um, im, usel, isel, w1a, w1b, b1r, w2, b2r, wpa, wpb, bpr)


def kernel(user_indices, item_indices, embed_user_GMF, embed_item_GMF,
           embed_user_MLP, embed_item_MLP, W1, b1, W2, b2, Wp, bp):
    ui = user_indices.astype(jnp.int32)
    ii = item_indices.astype(jnp.int32)
    u3 = (ui % _S).reshape(_NW, _NCHUNK, _CHUNK)
    i3 = (ii % _S).reshape(_NW, _NCHUNK, _CHUNK)
    tables = _tc_repack4(embed_user_GMF.T, embed_item_GMF.T,
                         embed_user_MLP.T, embed_item_MLP.T)
    ug, ig, um, im = _sc_gather4(u3, i3, *tables)
    ug = ug.reshape(_B, _LINE)
    ig = ig.reshape(_B, _LINE)
    um = um.reshape(_B, _LINE)
    im = im.reshape(_B, _LINE)
    usel = jax.nn.one_hot(ui // _S, _PACK, dtype=jnp.float32)
    isel = jax.nn.one_hot(ii // _S, _PACK, dtype=jnp.float32)
    w1a, w1b = W1[:_D], W1[_D:]
    wpa = Wp[:_D, 0].reshape(1, _D)
    wpb = Wp[_D:, 0].reshape(1, 16)
    out = _tc_mlp(ug, ig, um, im, usel, isel, w1a, w1b, b1.reshape(1, 32),
                  W2, b2.reshape(1, 16), wpa, wpb, bp.reshape(1, 1))
    return out.reshape(-1)
